# baseline (speedup 1.0000x reference)
import functools

import jax
import jax.numpy as jnp
from jax import lax
from jax.experimental import pallas as pl
from jax.experimental.pallas import tpu as pltpu
from jax.experimental.pallas import tpu_sc as plsc

N = 10000
E = 320000
CZ = 128
CE = 16
H = 8
C = 16
DIN = 2 * CZ + CE

NC = 2
NS = 16
NW = NC * NS
PW = 128
HH = H // NC
MW = HH * C

NHALF = 2
EH = E // NHALF
GCH = 40
SCH = 40
NP = 10240
NPS = NP // NS

_F32 = jnp.float32

_mesh = plsc.VectorSubcoreMesh(
    core_axis_name="c", subcore_axis_name="s", num_cores=NC, num_subcores=NS
)


_HW = CZ // 2


def _pack(v):
    a = lax.bitcast_convert_type(v[:, :_HW], jnp.int32)
    b = lax.bitcast_convert_type(v[:, _HW:], jnp.int32)
    a = jnp.bitwise_and(a + 0x8000, -65536)
    b = lax.shift_right_logical(b + 0x8000, 16)
    b = jnp.bitwise_and(b, 0xFFFF)
    return lax.bitcast_convert_type(jnp.bitwise_or(a, b), _F32)


def _unpack(p):
    w = lax.bitcast_convert_type(p, jnp.int32)
    hi = lax.bitcast_convert_type(jnp.bitwise_and(w, -65536), _F32)
    lo = lax.bitcast_convert_type(lax.shift_left(w, 16), _F32)
    return jnp.concatenate([hi, lo], axis=1)


def _node_tab_body(x_ref, wq_ref, bq_ref, out_ref):
    q = jnp.dot(x_ref[...], wq_ref[...], preferred_element_type=_F32)
    q = (q + bq_ref[...]) * 0.25
    out_ref[...] = jnp.concatenate([_pack(x_ref[...]), _pack(q)], axis=1)


def _node_tab(x, Wq, bq):
    bn = 2000
    return pl.pallas_call(
        _node_tab_body,
        grid=(N // bn,),
        in_specs=[
            pl.BlockSpec((bn, CZ), lambda i: (i, 0)),
            pl.BlockSpec((CZ, CZ), lambda i: (0, 0)),
            pl.BlockSpec((1, CZ), lambda i: (0, 0)),
        ],
        out_specs=pl.BlockSpec((bn, CZ), lambda i: (i, 0)),
        out_shape=jax.ShapeDtypeStruct((N, CZ), _F32),
    )(x, Wq, bq.reshape(1, CZ))


def _make_gather(ne, gch):
    epw = ne // NW
    gnch = epw // gch
    assert epw % gch == 0 and gnch % 2 == 1 and gch % 8 == 0

    def body(src_hbm, dst_hbm, xt_hbm, dt_hbm, g1_hbm, g2_hbm,
             sia, dia, b1a, b2a, sib, dib, b1b, b2b,
             sia_s, sga1, sga2, sib_s, sgb1, sgb2, swa, swb):
        wid = lax.axis_index("s") * NC + lax.axis_index("c")
        base = wid * epw

        def fire_idx(c, si, di, sem):
            off = base + c * gch
            pltpu.async_copy(src_hbm.at[pl.ds(off, gch)], si, sem)
            pltpu.async_copy(dst_hbm.at[pl.ds(off, gch)], di, sem)

        def drain_idx(c, si, di, sem):
            off = base + c * gch
            pltpu.make_async_copy(src_hbm.at[pl.ds(off, gch)], si, sem).wait()
            pltpu.make_async_copy(dst_hbm.at[pl.ds(off, gch)], di, sem).wait()

        def fire_rows(si, di, bb1, bb2, s1, s2):
            pltpu.async_copy(xt_hbm.at[si], bb1, s1)
            pltpu.async_copy(dt_hbm.at[di], bb2, s2)

        def drain_rows(si, di, bb1, bb2, s1, s2):
            pltpu.make_async_copy(xt_hbm.at[si], bb1, s1).wait()
            pltpu.make_async_copy(dt_hbm.at[di], bb2, s2).wait()

        def fire_wb(c, bb1, bb2, semw):
            off = base + c * gch
            pltpu.async_copy(bb1, g1_hbm.at[pl.ds(off, gch)], semw)
            pltpu.async_copy(bb2, g2_hbm.at[pl.ds(off, gch)], semw)

        def drain_wb(c, bb1, bb2, semw):
            off = base + c * gch
            pltpu.make_async_copy(bb1, g1_hbm.at[pl.ds(off, gch)], semw).wait()
            pltpu.make_async_copy(bb2, g2_hbm.at[pl.ds(off, gch)], semw).wait()

        seta = (sia, dia, b1a, b2a, sia_s, sga1, sga2, swa)
        setb = (sib, dib, b1b, b2b, sib_s, sgb1, sgb2, swb)

        def half_step(c, cur, nxt):
            si, di, bb1, bb2, sis, s1, s2, sw = cur
            nsi, ndi, nb1, nb2, nsis, ns1, ns2, nsw = nxt

            @pl.when(c > 0)
            def _():
                drain_wb(c - 1, nb1, nb2, nsw)

            fire_rows(nsi, ndi, nb1, nb2, ns1, ns2)
            drain_rows(si, di, bb1, bb2, s1, s2)
            fire_wb(c, bb1, bb2, sw)

            @pl.when(c + 2 < gnch)
            def _():
                fire_idx(c + 2, si, di, sis)
                drain_idx(c + 2, si, di, sis)

        fire_idx(0, sia, dia, sia_s)
        fire_idx(1, sib, dib, sib_s)
        drain_idx(0, sia, dia, sia_s)
        fire_rows(sia, dia, b1a, b2a, sga1, sga2)
        drain_idx(1, sib, dib, sib_s)

        def step(o, carry):
            c = 2 * o
            half_step(c, seta, setb)
            half_step(c + 1, setb, seta)
            return carry

        lax.fori_loop(0, (gnch - 1) // 2, step, 0)
        drain_rows(sia, dia, b1a, b2a, sga1, sga2)
        fire_wb(gnch - 1, b1a, b2a, swa)
        drain_wb(gnch - 2, b1b, b2b, swb)
        drain_wb(gnch - 1, b1a, b2a, swa)

    return functools.partial(
        pl.kernel,
        body,
        out_type=[
            jax.ShapeDtypeStruct((ne, CZ), _F32),
            jax.ShapeDtypeStruct((ne, CZ), _F32),
        ],
        mesh=_mesh,
        scratch_types=[
            pltpu.VMEM((gch,), jnp.int32),
            pltpu.VMEM((gch,), jnp.int32),
            pltpu.VMEM((gch, CZ), _F32),
            pltpu.VMEM((gch, CZ), _F32),
            pltpu.VMEM((gch,), jnp.int32),
            pltpu.VMEM((gch,), jnp.int32),
            pltpu.VMEM((gch, CZ), _F32),
            pltpu.VMEM((gch, CZ), _F32),
            pltpu.SemaphoreType.DMA,
            pltpu.SemaphoreType.DMA,
            pltpu.SemaphoreType.DMA,
            pltpu.SemaphoreType.DMA,
            pltpu.SemaphoreType.DMA,
            pltpu.SemaphoreType.DMA,
            pltpu.SemaphoreType.DMA,
            pltpu.SemaphoreType.DMA,
        ],
    )()


_gather_half = _make_gather(EH, GCH)


def _edge_body(ea_ref, g1_ref, g2_ref, lng_ref, lnb_ref,
               wk_ref, bk_ref, wv_ref, bv_ref, out_ref):
    ea = ea_ref[...]
    hi = g1_ref[...]
    d2 = lax.bitcast_convert_type(g2_ref[...], jnp.int32)
    xj = d2[:, :_HW]
    qj = d2[:, _HW:]
    hjA = lax.bitcast_convert_type(jnp.bitwise_and(xj, -65536), _F32)
    hjB = lax.bitcast_convert_type(lax.shift_left(xj, 16), _F32)
    qdA = lax.bitcast_convert_type(jnp.bitwise_and(qj, -65536), _F32)
    qdB = lax.bitcast_convert_type(lax.shift_left(qj, 16), _F32)

    def rsum(v):
        return jnp.sum(v, axis=1, keepdims=True)

    s1 = rsum(ea) + rsum(hi) + rsum(hjA) + rsum(hjB)
    s2 = (rsum(ea * ea) + rsum(hi * hi) + rsum(hjA * hjA)
          + rsum(hjB * hjB))
    mu = s1 * (1.0 / DIN)
    var = s2 * (1.0 / DIN) - mu * mu
    rs = lax.rsqrt(var + 1e-5)
    g = lng_ref[...]
    bb = lnb_ref[...]
    eag = ea * g[:, :CE]
    hig = hi * g[:, CE:CE + CZ]
    hjag = hjA * g[:, CE + CZ:CE + CZ + _HW]
    hjbg = hjB * g[:, CE + CZ + _HW:]

    def proj(w_ref, bx_ref):
        w = w_ref[...]
        core = (jnp.dot(eag, w[:CE], preferred_element_type=_F32)
                + jnp.dot(hig, w[CE:CE + CZ], preferred_element_type=_F32)
                + jnp.dot(hjag, w[CE + CZ:CE + CZ + _HW],
                          preferred_element_type=_F32)
                + jnp.dot(hjbg, w[CE + CZ + _HW:],
                          preferred_element_type=_F32))
        gk = jnp.dot(g, w, preferred_element_type=_F32)
        ck = jnp.dot(bb, w, preferred_element_type=_F32) + bx_ref[...]
        return rs * core - (mu * rs) * gk + ck

    key = proj(wk_ref, bk_ref)
    val = proj(wv_ref, bv_ref)
    hm = (lax.broadcasted_iota(jnp.int32, (_HW, HH), 0) // C
          == lax.broadcasted_iota(jnp.int32, (_HW, HH), 1)).astype(_F32)
    hmT = (lax.broadcasted_iota(jnp.int32, (HH, _HW), 1) // C
           == lax.broadcasted_iota(jnp.int32, (HH, _HW), 0)).astype(_F32)
    pad = jnp.zeros((ea.shape[0], PW - MW - HH), _F32)

    def half_rows(qd, keyh, valh):
        logits = jnp.dot(qd * keyh, hm, preferred_element_type=_F32)
        ex = jnp.exp(logits)
        m = valh * jnp.dot(ex, hmT, preferred_element_type=_F32)
        return jnp.concatenate([m, ex, pad], axis=1)

    row0 = half_rows(qdA, key[:, :_HW], val[:, :_HW])
    row1 = half_rows(qdB, key[:, _HW:], val[:, _HW:])
    out_ref[...] = jnp.stack([row0, row1], axis=0)


def _edge_math(edge_attr, g1, g2, ln_g, ln_b, Wk, bk, Wv, bv):
    be = 2000
    ne = g1.shape[0]
    return pl.pallas_call(
        _edge_body,
        grid=(ne // be,),
        in_specs=[
            pl.BlockSpec((be, CE), lambda i: (i, 0)),
            pl.BlockSpec((be, CZ), lambda i: (i, 0)),
            pl.BlockSpec((be, CZ), lambda i: (i, 0)),
            pl.BlockSpec((1, DIN), lambda i: (0, 0)),
            pl.BlockSpec((1, DIN), lambda i: (0, 0)),
            pl.BlockSpec((DIN, CZ), lambda i: (0, 0)),
            pl.BlockSpec((1, CZ), lambda i: (0, 0)),
            pl.BlockSpec((DIN, CZ), lambda i: (0, 0)),
            pl.BlockSpec((1, CZ), lambda i: (0, 0)),
        ],
        out_specs=pl.BlockSpec((NC, be, PW), lambda i: (0, i, 0)),
        out_shape=jax.ShapeDtypeStruct((NC, ne, PW), _F32),
    )(edge_attr, g1, g2, ln_g.reshape(1, DIN), ln_b.reshape(1, DIN),
      Wk, bk.reshape(1, CZ), Wv, bv.reshape(1, CZ))


_NSET = 4


def _make_scatter(ne, sch):
    eps = ne // NS
    snch = eps // sch
    tail = snch % _NSET
    assert eps % sch == 0 and sch % 8 == 0 and tail < _NSET

    def body(dst_hbm, pay_hbm, zer_hbm, out_hbm, *scr):
        dis = scr[0:_NSET]
        pbs = scr[_NSET:2 * _NSET]
        acc = scr[2 * _NSET]
        sls = scr[2 * _NSET + 1:3 * _NSET + 1]
        sas = scr[3 * _NSET + 1:4 * _NSET + 1]
        cid = lax.axis_index("c")
        sid = lax.axis_index("s")
        pltpu.sync_copy(zer_hbm, acc.at[pl.ds(sid * NPS, NPS)])
        plsc.subcore_barrier()
        base = sid * eps

        def fire_load(c, j):
            off = base + c * sch
            pltpu.async_copy(dst_hbm.at[pl.ds(off, sch)], dis[j], sls[j])
            pltpu.async_copy(pay_hbm.at[cid, pl.ds(off, sch)], pbs[j], sls[j])

        def drain_load(c, j):
            off = base + c * sch
            pltpu.make_async_copy(
                dst_hbm.at[pl.ds(off, sch)], dis[j], sls[j]).wait()
            pltpu.make_async_copy(
                pay_hbm.at[cid, pl.ds(off, sch)], pbs[j], sls[j]).wait()

        def fire_add(j):
            pltpu.async_copy(pbs[j], acc.at[dis[j]], sas[j], add=True)

        def drain_add(j):
            pltpu.make_async_copy(pbs[j], acc.at[dis[j]], sas[j]).wait()

        for j in range(_NSET):
            fire_load(j, j)

        def step(o, carry):
            c = _NSET * o
            for j in range(_NSET):
                drain_load(c + j, j)
                fire_add(j)
            for j in range(_NSET):
                drain_add(j)

                @pl.when(c + j + _NSET < snch)
                def _():
                    fire_load(c + j + _NSET, j)
            return carry

        lax.fori_loop(0, snch // _NSET, step, 0)
        for j in range(tail):
            c = snch - tail + j
            drain_load(c, j)
            fire_add(j)
        for j in range(tail):
            drain_add(j)
        plsc.subcore_barrier()
        pltpu.sync_copy(acc.at[pl.ds(sid * NPS, NPS)],
                        out_hbm.at[cid, pl.ds(sid * NPS, NPS)])

    return functools.partial(
        pl.kernel,
        body,
        out_type=jax.ShapeDtypeStruct((NC, NP, PW), _F32),
        mesh=_mesh,
        scratch_types=(
            [pltpu.VMEM((sch,), jnp.int32) for _ in range(_NSET)]
            + [pltpu.VMEM((sch, PW), _F32) for _ in range(_NSET)]
            + [pltpu.VMEM_SHARED((NP, PW), _F32)]
            + [pltpu.SemaphoreType.DMA for _ in range(2 * _NSET)]
        ),
    )()


_scatter_half = _make_scatter(EH, SCH)


def _final_body(p_ref, w1_ref, b1_ref, w2_ref, b2_ref, out_ref):
    p0 = p_ref[0] + p_ref[2]
    p1 = p_ref[1] + p_ref[3]
    msum = jnp.concatenate([p0[:, :MW], p1[:, :MW]], axis=1)
    den = jnp.concatenate([p0[:, MW:MW + HH], p1[:, MW:MW + HH]], axis=1)
    hmT = (lax.broadcasted_iota(jnp.int32, (H, CZ), 1) // C
           == lax.broadcasted_iota(jnp.int32, (H, CZ), 0)).astype(_F32)
    dexp = jnp.dot(den, hmT, preferred_element_type=_F32)
    out_x = msum / jnp.maximum(dexp, 1e-30)
    h = jnp.dot(out_x, w1_ref[...], preferred_element_type=_F32) + b1_ref[...]
    h = h * jax.nn.sigmoid(h)
    out_ref[...] = (jnp.dot(h, w2_ref[...], preferred_element_type=_F32)
                    + b2_ref[...])


def _final(parts, W1, b1, W2, b2):
    bn = 2000
    return pl.pallas_call(
        _final_body,
        grid=(N // bn,),
        in_specs=[
            pl.BlockSpec((2 * NC, bn, PW), lambda i: (0, i, 0)),
            pl.BlockSpec((CZ, 2 * CZ), lambda i: (0, 0)),
            pl.BlockSpec((1, 2 * CZ), lambda i: (0, 0)),
            pl.BlockSpec((2 * CZ, CZ), lambda i: (0, 0)),
            pl.BlockSpec((1, CZ), lambda i: (0, 0)),
        ],
        out_specs=pl.BlockSpec((bn, CZ), lambda i: (i, 0)),
        out_shape=jax.ShapeDtypeStruct((N, CZ), _F32),
    )(parts, W1, b1.reshape(1, 2 * CZ), W2, b2.reshape(1, CZ))


def kernel(x, edge_index, edge_attr, ln_g, ln_b, Wq, bq, Wk, bk, Wv, bv,
           W1, b1, W2, b2):
    src = edge_index[0]
    dst = edge_index[1]
    dtab = _node_tab(x, Wq, bq)
    zeros = jnp.zeros((NPS, PW), _F32)
    parts = []
    for hf in range(NHALF):
        sl = slice(hf * EH, (hf + 1) * EH)
        g1, g2 = _gather_half(src[sl], dst[sl], x, dtab)
        pay = _edge_math(edge_attr[sl], g1, g2, ln_g, ln_b, Wk, bk, Wv, bv)
        parts.append(_scatter_half(dst[sl], pay, zeros))
    stacked = jnp.concatenate([parts[0], parts[1]], axis=0)
    return _final(stacked, W1, b1, W2, b2)


# speedup vs baseline: 1.2521x; 1.2521x over previous
import functools

import jax
import jax.numpy as jnp
from jax import lax
from jax.experimental import pallas as pl
from jax.experimental.pallas import tpu as pltpu
from jax.experimental.pallas import tpu_sc as plsc

N = 10000
E = 320000
CZ = 128
CE = 16
H = 8
C = 16
DIN = 2 * CZ + CE

NC = 2
NS = 16
NW = NC * NS
PW = 128
HH = H // NC
MW = HH * C

NHALF = 2
EH = E // NHALF
GCH = 40
SCH = 40
NP = 10240
NPS = NP // NS

_F32 = jnp.float32

_mesh = plsc.VectorSubcoreMesh(
    core_axis_name="c", subcore_axis_name="s", num_cores=NC, num_subcores=NS
)


_HW = CZ // 2


def _pack(v):
    a = lax.bitcast_convert_type(v[:, :_HW], jnp.int32)
    b = lax.bitcast_convert_type(v[:, _HW:], jnp.int32)
    a = jnp.bitwise_and(a + 0x8000, -65536)
    b = lax.shift_right_logical(b + 0x8000, 16)
    b = jnp.bitwise_and(b, 0xFFFF)
    return lax.bitcast_convert_type(jnp.bitwise_or(a, b), _F32)


def _unpack(p):
    w = lax.bitcast_convert_type(p, jnp.int32)
    hi = lax.bitcast_convert_type(jnp.bitwise_and(w, -65536), _F32)
    lo = lax.bitcast_convert_type(lax.shift_left(w, 16), _F32)
    return jnp.concatenate([hi, lo], axis=1)


def _node_tab_body(x_ref, wq_ref, bq_ref, out_ref):
    q = jnp.dot(x_ref[...], wq_ref[...], preferred_element_type=_F32)
    q = (q + bq_ref[...]) * 0.25
    out_ref[...] = jnp.concatenate([_pack(x_ref[...]), _pack(q)], axis=1)


def _node_tab(x, Wq, bq):
    bn = 2000
    return pl.pallas_call(
        _node_tab_body,
        grid=(N // bn,),
        in_specs=[
            pl.BlockSpec((bn, CZ), lambda i: (i, 0)),
            pl.BlockSpec((CZ, CZ), lambda i: (0, 0)),
            pl.BlockSpec((1, CZ), lambda i: (0, 0)),
        ],
        out_specs=pl.BlockSpec((bn, CZ), lambda i: (i, 0)),
        out_shape=jax.ShapeDtypeStruct((N, CZ), _F32),
    )(x, Wq, bq.reshape(1, CZ))


def _make_gather(ne, gch):
    epw = ne // NW
    gnch = epw // gch
    assert epw % gch == 0 and gnch % 2 == 1 and gch % 8 == 0

    def body(src_hbm, dst_hbm, xt_hbm, dt_hbm, g1_hbm, g2_hbm,
             sia, dia, b1a, b2a, sib, dib, b1b, b2b,
             sia_s, sga1, sga2, sib_s, sgb1, sgb2, swa, swb):
        wid = lax.axis_index("s") * NC + lax.axis_index("c")
        base = wid * epw

        def fire_idx(c, si, di, sem):
            off = base + c * gch
            pltpu.async_copy(src_hbm.at[pl.ds(off, gch)], si, sem)
            pltpu.async_copy(dst_hbm.at[pl.ds(off, gch)], di, sem)

        def drain_idx(c, si, di, sem):
            off = base + c * gch
            pltpu.make_async_copy(src_hbm.at[pl.ds(off, gch)], si, sem).wait()
            pltpu.make_async_copy(dst_hbm.at[pl.ds(off, gch)], di, sem).wait()

        def fire_rows(si, di, bb1, bb2, s1, s2):
            pltpu.async_copy(xt_hbm.at[si], bb1, s1)
            pltpu.async_copy(dt_hbm.at[di], bb2, s2)

        def drain_rows(si, di, bb1, bb2, s1, s2):
            pltpu.make_async_copy(xt_hbm.at[si], bb1, s1).wait()
            pltpu.make_async_copy(dt_hbm.at[di], bb2, s2).wait()

        def fire_wb(c, bb1, bb2, semw):
            off = base + c * gch
            pltpu.async_copy(bb1, g1_hbm.at[pl.ds(off, gch)], semw)
            pltpu.async_copy(bb2, g2_hbm.at[pl.ds(off, gch)], semw)

        def drain_wb(c, bb1, bb2, semw):
            off = base + c * gch
            pltpu.make_async_copy(bb1, g1_hbm.at[pl.ds(off, gch)], semw).wait()
            pltpu.make_async_copy(bb2, g2_hbm.at[pl.ds(off, gch)], semw).wait()

        seta = (sia, dia, b1a, b2a, sia_s, sga1, sga2, swa)
        setb = (sib, dib, b1b, b2b, sib_s, sgb1, sgb2, swb)

        def half_step(c, cur, nxt):
            si, di, bb1, bb2, sis, s1, s2, sw = cur
            nsi, ndi, nb1, nb2, nsis, ns1, ns2, nsw = nxt

            @pl.when(c > 0)
            def _():
                drain_wb(c - 1, nb1, nb2, nsw)

            fire_rows(nsi, ndi, nb1, nb2, ns1, ns2)
            drain_rows(si, di, bb1, bb2, s1, s2)
            fire_wb(c, bb1, bb2, sw)

            @pl.when(c + 2 < gnch)
            def _():
                fire_idx(c + 2, si, di, sis)
                drain_idx(c + 2, si, di, sis)

        fire_idx(0, sia, dia, sia_s)
        fire_idx(1, sib, dib, sib_s)
        drain_idx(0, sia, dia, sia_s)
        fire_rows(sia, dia, b1a, b2a, sga1, sga2)
        drain_idx(1, sib, dib, sib_s)

        def step(o, carry):
            c = 2 * o
            half_step(c, seta, setb)
            half_step(c + 1, setb, seta)
            return carry

        lax.fori_loop(0, (gnch - 1) // 2, step, 0)
        drain_rows(sia, dia, b1a, b2a, sga1, sga2)
        fire_wb(gnch - 1, b1a, b2a, swa)
        drain_wb(gnch - 2, b1b, b2b, swb)
        drain_wb(gnch - 1, b1a, b2a, swa)

    return functools.partial(
        pl.kernel,
        body,
        out_type=[
            jax.ShapeDtypeStruct((ne, CZ), _F32),
            jax.ShapeDtypeStruct((ne, CZ), _F32),
        ],
        mesh=_mesh,
        scratch_types=[
            pltpu.VMEM((gch,), jnp.int32),
            pltpu.VMEM((gch,), jnp.int32),
            pltpu.VMEM((gch, CZ), _F32),
            pltpu.VMEM((gch, CZ), _F32),
            pltpu.VMEM((gch,), jnp.int32),
            pltpu.VMEM((gch,), jnp.int32),
            pltpu.VMEM((gch, CZ), _F32),
            pltpu.VMEM((gch, CZ), _F32),
            pltpu.SemaphoreType.DMA,
            pltpu.SemaphoreType.DMA,
            pltpu.SemaphoreType.DMA,
            pltpu.SemaphoreType.DMA,
            pltpu.SemaphoreType.DMA,
            pltpu.SemaphoreType.DMA,
            pltpu.SemaphoreType.DMA,
            pltpu.SemaphoreType.DMA,
        ],
    )()


_gather_half = _make_gather(EH, GCH)


def _edge_body(ea_ref, g1_ref, g2_ref, lng_ref, lnb_ref,
               wk_ref, bk_ref, wv_ref, bv_ref, out_ref):
    hi = g1_ref[...]
    d2 = g2_ref[...]
    hj = _unpack(d2[:, :_HW])
    qd = _unpack(d2[:, _HW:])
    xf = jnp.concatenate([ea_ref[...], hi, hj], axis=1)
    mu = jnp.mean(xf, axis=1, keepdims=True)
    xc = xf - mu
    var = jnp.mean(xc * xc, axis=1, keepdims=True)
    xn = xc * lax.rsqrt(var + 1e-5) * lng_ref[...] + lnb_ref[...]
    key = jnp.dot(xn, wk_ref[...], preferred_element_type=_F32) + bk_ref[...]
    val = jnp.dot(xn, wv_ref[...], preferred_element_type=_F32) + bv_ref[...]
    hm = (lax.broadcasted_iota(jnp.int32, (CZ, H), 0) // C
          == lax.broadcasted_iota(jnp.int32, (CZ, H), 1)).astype(_F32)
    hmT = (lax.broadcasted_iota(jnp.int32, (H, CZ), 1) // C
           == lax.broadcasted_iota(jnp.int32, (H, CZ), 0)).astype(_F32)
    logits = jnp.dot(qd * key, hm, preferred_element_type=_F32)
    ex = jnp.exp(logits)
    m = val * jnp.dot(ex, hmT, preferred_element_type=_F32)
    pad = jnp.zeros((m.shape[0], PW - MW - HH), _F32)
    row0 = jnp.concatenate([m[:, :MW], ex[:, :HH], pad], axis=1)
    row1 = jnp.concatenate([m[:, MW:], ex[:, HH:], pad], axis=1)
    out_ref[...] = jnp.stack([row0, row1], axis=0)


def _edge_math(edge_attr, g1, g2, ln_g, ln_b, Wk, bk, Wv, bv):
    be = 2000
    ne = g1.shape[0]
    return pl.pallas_call(
        _edge_body,
        grid=(ne // be,),
        in_specs=[
            pl.BlockSpec((be, CE), lambda i: (i, 0)),
            pl.BlockSpec((be, CZ), lambda i: (i, 0)),
            pl.BlockSpec((be, CZ), lambda i: (i, 0)),
            pl.BlockSpec((1, DIN), lambda i: (0, 0)),
            pl.BlockSpec((1, DIN), lambda i: (0, 0)),
            pl.BlockSpec((DIN, CZ), lambda i: (0, 0)),
            pl.BlockSpec((1, CZ), lambda i: (0, 0)),
            pl.BlockSpec((DIN, CZ), lambda i: (0, 0)),
            pl.BlockSpec((1, CZ), lambda i: (0, 0)),
        ],
        out_specs=pl.BlockSpec((NC, be, PW), lambda i: (0, i, 0)),
        out_shape=jax.ShapeDtypeStruct((NC, ne, PW), _F32),
    )(edge_attr, g1, g2, ln_g.reshape(1, DIN), ln_b.reshape(1, DIN),
      Wk, bk.reshape(1, CZ), Wv, bv.reshape(1, CZ))


_NSET = 4


def _make_scatter(ne, sch):
    eps = ne // NS
    snch = eps // sch
    tail = snch % _NSET
    assert eps % sch == 0 and sch % 8 == 0 and tail < _NSET

    def body(dst_hbm, pay_hbm, zer_hbm, out_hbm, *scr):
        dis = scr[0:_NSET]
        pbs = scr[_NSET:2 * _NSET]
        acc = scr[2 * _NSET]
        sls = scr[2 * _NSET + 1:3 * _NSET + 1]
        sas = scr[3 * _NSET + 1:4 * _NSET + 1]
        cid = lax.axis_index("c")
        sid = lax.axis_index("s")
        pltpu.sync_copy(zer_hbm, acc.at[pl.ds(sid * NPS, NPS)])
        plsc.subcore_barrier()
        base = sid * eps

        def fire_load(c, j):
            off = base + c * sch
            pltpu.async_copy(dst_hbm.at[pl.ds(off, sch)], dis[j], sls[j])
            pltpu.async_copy(pay_hbm.at[cid, pl.ds(off, sch)], pbs[j], sls[j])

        def drain_load(c, j):
            off = base + c * sch
            pltpu.make_async_copy(
                dst_hbm.at[pl.ds(off, sch)], dis[j], sls[j]).wait()
            pltpu.make_async_copy(
                pay_hbm.at[cid, pl.ds(off, sch)], pbs[j], sls[j]).wait()

        def fire_add(j):
            pltpu.async_copy(pbs[j], acc.at[dis[j]], sas[j], add=True)

        def drain_add(j):
            pltpu.make_async_copy(pbs[j], acc.at[dis[j]], sas[j]).wait()

        for j in range(_NSET):
            fire_load(j, j)

        def step(o, carry):
            c = _NSET * o
            for j in range(_NSET):
                drain_load(c + j, j)
                fire_add(j)
            for j in range(_NSET):
                drain_add(j)

                @pl.when(c + j + _NSET < snch)
                def _():
                    fire_load(c + j + _NSET, j)
            return carry

        lax.fori_loop(0, snch // _NSET, step, 0)
        for j in range(tail):
            c = snch - tail + j
            drain_load(c, j)
            fire_add(j)
        for j in range(tail):
            drain_add(j)
        plsc.subcore_barrier()
        pltpu.sync_copy(acc.at[pl.ds(sid * NPS, NPS)],
                        out_hbm.at[cid, pl.ds(sid * NPS, NPS)])

    return functools.partial(
        pl.kernel,
        body,
        out_type=jax.ShapeDtypeStruct((NC, NP, PW), _F32),
        mesh=_mesh,
        scratch_types=(
            [pltpu.VMEM((sch,), jnp.int32) for _ in range(_NSET)]
            + [pltpu.VMEM((sch, PW), _F32) for _ in range(_NSET)]
            + [pltpu.VMEM_SHARED((NP, PW), _F32)]
            + [pltpu.SemaphoreType.DMA for _ in range(2 * _NSET)]
        ),
    )()


_scatter_half = _make_scatter(EH, SCH)


def _final_body(p_ref, w1_ref, b1_ref, w2_ref, b2_ref, out_ref):
    p0 = p_ref[0] + p_ref[2]
    p1 = p_ref[1] + p_ref[3]
    msum = jnp.concatenate([p0[:, :MW], p1[:, :MW]], axis=1)
    den = jnp.concatenate([p0[:, MW:MW + HH], p1[:, MW:MW + HH]], axis=1)
    hmT = (lax.broadcasted_iota(jnp.int32, (H, CZ), 1) // C
           == lax.broadcasted_iota(jnp.int32, (H, CZ), 0)).astype(_F32)
    dexp = jnp.dot(den, hmT, preferred_element_type=_F32)
    out_x = msum / jnp.maximum(dexp, 1e-30)
    h = jnp.dot(out_x, w1_ref[...], preferred_element_type=_F32) + b1_ref[...]
    h = h * jax.nn.sigmoid(h)
    out_ref[...] = (jnp.dot(h, w2_ref[...], preferred_element_type=_F32)
                    + b2_ref[...])


def _final(parts, W1, b1, W2, b2):
    bn = 2000
    return pl.pallas_call(
        _final_body,
        grid=(N // bn,),
        in_specs=[
            pl.BlockSpec((2 * NC, bn, PW), lambda i: (0, i, 0)),
            pl.BlockSpec((CZ, 2 * CZ), lambda i: (0, 0)),
            pl.BlockSpec((1, 2 * CZ), lambda i: (0, 0)),
            pl.BlockSpec((2 * CZ, CZ), lambda i: (0, 0)),
            pl.BlockSpec((1, CZ), lambda i: (0, 0)),
        ],
        out_specs=pl.BlockSpec((bn, CZ), lambda i: (i, 0)),
        out_shape=jax.ShapeDtypeStruct((N, CZ), _F32),
    )(parts, W1, b1.reshape(1, 2 * CZ), W2, b2.reshape(1, CZ))


def kernel(x, edge_index, edge_attr, ln_g, ln_b, Wq, bq, Wk, bk, Wv, bv,
           W1, b1, W2, b2):
    src = edge_index[0]
    dst = edge_index[1]
    dtab = _node_tab(x, Wq, bq)
    zeros = jnp.zeros((NPS, PW), _F32)
    parts = []
    for hf in range(NHALF):
        sl = slice(hf * EH, (hf + 1) * EH)
        g1, g2 = _gather_half(src[sl], dst[sl], x, dtab)
        pay = _edge_math(edge_attr[sl], g1, g2, ln_g, ln_b, Wk, bk, Wv, bv)
        parts.append(_scatter_half(dst[sl], pay, zeros))
    stacked = jnp.concatenate([parts[0], parts[1]], axis=0)
    return _final(stacked, W1, b1, W2, b2)


# speedup vs baseline: 1.3209x; 1.0550x over previous
import functools

import jax
import jax.numpy as jnp
from jax import lax
from jax.experimental import pallas as pl
from jax.experimental.pallas import tpu as pltpu
from jax.experimental.pallas import tpu_sc as plsc

N = 10000
E = 320000
CZ = 128
CE = 16
H = 8
C = 16
DIN = 2 * CZ + CE

NC = 2
NS = 16
NW = NC * NS
PW = 128
HH = H // NC
MW = HH * C

NHALF = 2
EH = E // NHALF
GCH = 40
SCH = 80
NP = 10240
NPS = NP // NS

_F32 = jnp.float32

_mesh = plsc.VectorSubcoreMesh(
    core_axis_name="c", subcore_axis_name="s", num_cores=NC, num_subcores=NS
)


_HW = CZ // 2


def _pack(v):
    a = lax.bitcast_convert_type(v[:, :_HW], jnp.int32)
    b = lax.bitcast_convert_type(v[:, _HW:], jnp.int32)
    a = jnp.bitwise_and(a + 0x8000, -65536)
    b = lax.shift_right_logical(b + 0x8000, 16)
    b = jnp.bitwise_and(b, 0xFFFF)
    return lax.bitcast_convert_type(jnp.bitwise_or(a, b), _F32)


def _unpack(p):
    w = lax.bitcast_convert_type(p, jnp.int32)
    hi = lax.bitcast_convert_type(jnp.bitwise_and(w, -65536), _F32)
    lo = lax.bitcast_convert_type(lax.shift_left(w, 16), _F32)
    return jnp.concatenate([hi, lo], axis=1)


def _node_tab_body(x_ref, wq_ref, bq_ref, out_ref):
    q = jnp.dot(x_ref[...], wq_ref[...], preferred_element_type=_F32)
    q = (q + bq_ref[...]) * 0.25
    out_ref[...] = jnp.concatenate([_pack(x_ref[...]), _pack(q)], axis=1)


def _node_tab(x, Wq, bq):
    bn = 2000
    return pl.pallas_call(
        _node_tab_body,
        grid=(N // bn,),
        in_specs=[
            pl.BlockSpec((bn, CZ), lambda i: (i, 0)),
            pl.BlockSpec((CZ, CZ), lambda i: (0, 0)),
            pl.BlockSpec((1, CZ), lambda i: (0, 0)),
        ],
        out_specs=pl.BlockSpec((bn, CZ), lambda i: (i, 0)),
        out_shape=jax.ShapeDtypeStruct((N, CZ), _F32),
    )(x, Wq, bq.reshape(1, CZ))


def _make_gather(ne, gch):
    epw = ne // NW
    gnch = epw // gch
    assert epw % gch == 0 and gnch % 2 == 1 and gch % 8 == 0

    def body(src_hbm, dst_hbm, xt_hbm, dt_hbm, g1_hbm, g2_hbm,
             sia, dia, b1a, b2a, sib, dib, b1b, b2b,
             sia_s, sga1, sga2, sib_s, sgb1, sgb2, swa, swb):
        wid = lax.axis_index("s") * NC + lax.axis_index("c")
        base = wid * epw

        def fire_idx(c, si, di, sem):
            off = base + c * gch
            pltpu.async_copy(src_hbm.at[pl.ds(off, gch)], si, sem)
            pltpu.async_copy(dst_hbm.at[pl.ds(off, gch)], di, sem)

        def drain_idx(c, si, di, sem):
            off = base + c * gch
            pltpu.make_async_copy(src_hbm.at[pl.ds(off, gch)], si, sem).wait()
            pltpu.make_async_copy(dst_hbm.at[pl.ds(off, gch)], di, sem).wait()

        def fire_rows(si, di, bb1, bb2, s1, s2):
            pltpu.async_copy(xt_hbm.at[si], bb1, s1)
            pltpu.async_copy(dt_hbm.at[di], bb2, s2)

        def drain_rows(si, di, bb1, bb2, s1, s2):
            pltpu.make_async_copy(xt_hbm.at[si], bb1, s1).wait()
            pltpu.make_async_copy(dt_hbm.at[di], bb2, s2).wait()

        def fire_wb(c, bb1, bb2, semw):
            off = base + c * gch
            pltpu.async_copy(bb1, g1_hbm.at[pl.ds(off, gch)], semw)
            pltpu.async_copy(bb2, g2_hbm.at[pl.ds(off, gch)], semw)

        def drain_wb(c, bb1, bb2, semw):
            off = base + c * gch
            pltpu.make_async_copy(bb1, g1_hbm.at[pl.ds(off, gch)], semw).wait()
            pltpu.make_async_copy(bb2, g2_hbm.at[pl.ds(off, gch)], semw).wait()

        seta = (sia, dia, b1a, b2a, sia_s, sga1, sga2, swa)
        setb = (sib, dib, b1b, b2b, sib_s, sgb1, sgb2, swb)

        def half_step(c, cur, nxt):
            si, di, bb1, bb2, sis, s1, s2, sw = cur
            nsi, ndi, nb1, nb2, nsis, ns1, ns2, nsw = nxt

            @pl.when(c > 0)
            def _():
                drain_wb(c - 1, nb1, nb2, nsw)

            fire_rows(nsi, ndi, nb1, nb2, ns1, ns2)
            drain_rows(si, di, bb1, bb2, s1, s2)
            fire_wb(c, bb1, bb2, sw)

            @pl.when(c + 2 < gnch)
            def _():
                fire_idx(c + 2, si, di, sis)
                drain_idx(c + 2, si, di, sis)

        fire_idx(0, sia, dia, sia_s)
        fire_idx(1, sib, dib, sib_s)
        drain_idx(0, sia, dia, sia_s)
        fire_rows(sia, dia, b1a, b2a, sga1, sga2)
        drain_idx(1, sib, dib, sib_s)

        def step(o, carry):
            c = 2 * o
            half_step(c, seta, setb)
            half_step(c + 1, setb, seta)
            return carry

        lax.fori_loop(0, (gnch - 1) // 2, step, 0)
        drain_rows(sia, dia, b1a, b2a, sga1, sga2)
        fire_wb(gnch - 1, b1a, b2a, swa)
        drain_wb(gnch - 2, b1b, b2b, swb)
        drain_wb(gnch - 1, b1a, b2a, swa)

    return functools.partial(
        pl.kernel,
        body,
        out_type=[
            jax.ShapeDtypeStruct((ne, CZ), _F32),
            jax.ShapeDtypeStruct((ne, CZ), _F32),
        ],
        mesh=_mesh,
        scratch_types=[
            pltpu.VMEM((gch,), jnp.int32),
            pltpu.VMEM((gch,), jnp.int32),
            pltpu.VMEM((gch, CZ), _F32),
            pltpu.VMEM((gch, CZ), _F32),
            pltpu.VMEM((gch,), jnp.int32),
            pltpu.VMEM((gch,), jnp.int32),
            pltpu.VMEM((gch, CZ), _F32),
            pltpu.VMEM((gch, CZ), _F32),
            pltpu.SemaphoreType.DMA,
            pltpu.SemaphoreType.DMA,
            pltpu.SemaphoreType.DMA,
            pltpu.SemaphoreType.DMA,
            pltpu.SemaphoreType.DMA,
            pltpu.SemaphoreType.DMA,
            pltpu.SemaphoreType.DMA,
            pltpu.SemaphoreType.DMA,
        ],
    )()


_gather_half = _make_gather(EH, GCH)


def _edge_body(ea_ref, g1_ref, g2_ref, lng_ref, lnb_ref,
               wk_ref, bk_ref, wv_ref, bv_ref, out_ref):
    hi = g1_ref[...]
    d2 = g2_ref[...]
    hj = _unpack(d2[:, :_HW])
    qd = _unpack(d2[:, _HW:])
    xf = jnp.concatenate([ea_ref[...], hi, hj], axis=1)
    mu = jnp.mean(xf, axis=1, keepdims=True)
    xc = xf - mu
    var = jnp.mean(xc * xc, axis=1, keepdims=True)
    xn = xc * lax.rsqrt(var + 1e-5) * lng_ref[...] + lnb_ref[...]
    key = jnp.dot(xn, wk_ref[...], preferred_element_type=_F32) + bk_ref[...]
    val = jnp.dot(xn, wv_ref[...], preferred_element_type=_F32) + bv_ref[...]
    hm = (lax.broadcasted_iota(jnp.int32, (CZ, H), 0) // C
          == lax.broadcasted_iota(jnp.int32, (CZ, H), 1)).astype(_F32)
    hmT = (lax.broadcasted_iota(jnp.int32, (H, CZ), 1) // C
           == lax.broadcasted_iota(jnp.int32, (H, CZ), 0)).astype(_F32)
    logits = jnp.dot(qd * key, hm, preferred_element_type=_F32)
    ex = jnp.exp(logits)
    m = val * jnp.dot(ex, hmT, preferred_element_type=_F32)
    pad = jnp.zeros((m.shape[0], PW - MW - HH), _F32)
    row0 = jnp.concatenate([m[:, :MW], ex[:, :HH], pad], axis=1)
    row1 = jnp.concatenate([m[:, MW:], ex[:, HH:], pad], axis=1)
    out_ref[...] = jnp.stack([row0, row1], axis=0)


def _edge_math(edge_attr, g1, g2, ln_g, ln_b, Wk, bk, Wv, bv):
    be = 4000
    ne = g1.shape[0]
    return pl.pallas_call(
        _edge_body,
        grid=(ne // be,),
        in_specs=[
            pl.BlockSpec((be, CE), lambda i: (i, 0)),
            pl.BlockSpec((be, CZ), lambda i: (i, 0)),
            pl.BlockSpec((be, CZ), lambda i: (i, 0)),
            pl.BlockSpec((1, DIN), lambda i: (0, 0)),
            pl.BlockSpec((1, DIN), lambda i: (0, 0)),
            pl.BlockSpec((DIN, CZ), lambda i: (0, 0)),
            pl.BlockSpec((1, CZ), lambda i: (0, 0)),
            pl.BlockSpec((DIN, CZ), lambda i: (0, 0)),
            pl.BlockSpec((1, CZ), lambda i: (0, 0)),
        ],
        out_specs=pl.BlockSpec((NC, be, PW), lambda i: (0, i, 0)),
        out_shape=jax.ShapeDtypeStruct((NC, ne, PW), _F32),
    )(edge_attr, g1, g2, ln_g.reshape(1, DIN), ln_b.reshape(1, DIN),
      Wk, bk.reshape(1, CZ), Wv, bv.reshape(1, CZ))


_NSET = 4


def _make_scatter(ne, sch):
    eps = ne // NS
    snch = eps // sch
    tail = snch % _NSET
    assert eps % sch == 0 and sch % 8 == 0 and tail < _NSET

    def body(dst_hbm, pay_hbm, zer_hbm, out_hbm, *scr):
        dis = scr[0:_NSET]
        pbs = scr[_NSET:2 * _NSET]
        acc = scr[2 * _NSET]
        sls = scr[2 * _NSET + 1:3 * _NSET + 1]
        sas = scr[3 * _NSET + 1:4 * _NSET + 1]
        cid = lax.axis_index("c")
        sid = lax.axis_index("s")
        pltpu.sync_copy(zer_hbm, acc.at[pl.ds(sid * NPS, NPS)])
        plsc.subcore_barrier()
        base = sid * eps

        def fire_load(c, j):
            off = base + c * sch
            pltpu.async_copy(dst_hbm.at[pl.ds(off, sch)], dis[j], sls[j])
            pltpu.async_copy(pay_hbm.at[cid, pl.ds(off, sch)], pbs[j], sls[j])

        def drain_load(c, j):
            off = base + c * sch
            pltpu.make_async_copy(
                dst_hbm.at[pl.ds(off, sch)], dis[j], sls[j]).wait()
            pltpu.make_async_copy(
                pay_hbm.at[cid, pl.ds(off, sch)], pbs[j], sls[j]).wait()

        def fire_add(j):
            pltpu.async_copy(pbs[j], acc.at[dis[j]], sas[j], add=True)

        def drain_add(j):
            pltpu.make_async_copy(pbs[j], acc.at[dis[j]], sas[j]).wait()

        for j in range(_NSET):
            fire_load(j, j)

        def step(o, carry):
            c = _NSET * o
            for j in range(_NSET):
                drain_load(c + j, j)
                fire_add(j)
            for j in range(_NSET):
                drain_add(j)

                @pl.when(c + j + _NSET < snch)
                def _():
                    fire_load(c + j + _NSET, j)
            return carry

        lax.fori_loop(0, snch // _NSET, step, 0)
        for j in range(tail):
            c = snch - tail + j
            drain_load(c, j)
            fire_add(j)
        for j in range(tail):
            drain_add(j)
        plsc.subcore_barrier()
        pltpu.sync_copy(acc.at[pl.ds(sid * NPS, NPS)],
                        out_hbm.at[cid, pl.ds(sid * NPS, NPS)])

    return functools.partial(
        pl.kernel,
        body,
        out_type=jax.ShapeDtypeStruct((NC, NP, PW), _F32),
        mesh=_mesh,
        scratch_types=(
            [pltpu.VMEM((sch,), jnp.int32) for _ in range(_NSET)]
            + [pltpu.VMEM((sch, PW), _F32) for _ in range(_NSET)]
            + [pltpu.VMEM_SHARED((NP, PW), _F32)]
            + [pltpu.SemaphoreType.DMA for _ in range(2 * _NSET)]
        ),
    )()


_scatter_half = _make_scatter(EH, SCH)


def _final_body(p_ref, w1_ref, b1_ref, w2_ref, b2_ref, out_ref):
    p0 = p_ref[0] + p_ref[2]
    p1 = p_ref[1] + p_ref[3]
    msum = jnp.concatenate([p0[:, :MW], p1[:, :MW]], axis=1)
    den = jnp.concatenate([p0[:, MW:MW + HH], p1[:, MW:MW + HH]], axis=1)
    hmT = (lax.broadcasted_iota(jnp.int32, (H, CZ), 1) // C
           == lax.broadcasted_iota(jnp.int32, (H, CZ), 0)).astype(_F32)
    dexp = jnp.dot(den, hmT, preferred_element_type=_F32)
    out_x = msum / jnp.maximum(dexp, 1e-30)
    h = jnp.dot(out_x, w1_ref[...], preferred_element_type=_F32) + b1_ref[...]
    h = h * jax.nn.sigmoid(h)
    out_ref[...] = (jnp.dot(h, w2_ref[...], preferred_element_type=_F32)
                    + b2_ref[...])


def _final(parts, W1, b1, W2, b2):
    bn = 2000
    return pl.pallas_call(
        _final_body,
        grid=(N // bn,),
        in_specs=[
            pl.BlockSpec((2 * NC, bn, PW), lambda i: (0, i, 0)),
            pl.BlockSpec((CZ, 2 * CZ), lambda i: (0, 0)),
            pl.BlockSpec((1, 2 * CZ), lambda i: (0, 0)),
            pl.BlockSpec((2 * CZ, CZ), lambda i: (0, 0)),
            pl.BlockSpec((1, CZ), lambda i: (0, 0)),
        ],
        out_specs=pl.BlockSpec((bn, CZ), lambda i: (i, 0)),
        out_shape=jax.ShapeDtypeStruct((N, CZ), _F32),
    )(parts, W1, b1.reshape(1, 2 * CZ), W2, b2.reshape(1, CZ))


def kernel(x, edge_index, edge_attr, ln_g, ln_b, Wq, bq, Wk, bk, Wv, bv,
           W1, b1, W2, b2):
    src = edge_index[0]
    dst = edge_index[1]
    dtab = _node_tab(x, Wq, bq)
    zeros = jnp.zeros((NPS, PW), _F32)
    parts = []
    for hf in range(NHALF):
        sl = slice(hf * EH, (hf + 1) * EH)
        g1, g2 = _gather_half(src[sl], dst[sl], x, dtab)
        pay = _edge_math(edge_attr[sl], g1, g2, ln_g, ln_b, Wk, bk, Wv, bv)
        parts.append(_scatter_half(dst[sl], pay, zeros))
    stacked = jnp.concatenate([parts[0], parts[1]], axis=0)
    return _final(stacked, W1, b1, W2, b2)
